# Initial kernel scaffold; baseline (speedup 1.0000x reference)
import jax
import jax.numpy as jnp
from jax.experimental import pallas as pl


def kernel(x, edge_index, Wp, bp, Wconv, bconv, gamma, beta, layer_weights, Wc1, bc1, Wc2, bc2, Wc3, bc3):
    raise NotImplementedError("write your pallas kernel here")


# speedup vs baseline: 9.0452x; 9.0452x over previous
import functools

import jax
import jax.numpy as jnp
from jax import lax
from jax.experimental import pallas as pl
from jax.experimental.pallas import tpu as pltpu
from jax.experimental.pallas import tpu_sc as plsc

N = 10000
E = 320000
D = 128
H = 128
C = 2
EPS = 1e-5

NC = 2
NS = 16
NW = NC * NS

NPAD = 10240
SR = NPAD // NS
CH = 128
EEPAD = 323584
ROWS = EEPAD // CH
RPW = ROWS // NW

_mesh = plsc.VectorSubcoreMesh(
    core_axis_name="c", subcore_axis_name="s", num_cores=NC, num_subcores=NS)


def _fill_zero_rows(zrow_v):
    z16 = jnp.zeros((16,), jnp.float32)
    for r in range(8):
        for k in range(8):
            zrow_v[r, pl.ds(k * 16, 16)] = z16


def _make_agg(with_counts):
    out_type = [jax.ShapeDtypeStruct((NC, NPAD, H), jnp.float32)]
    scratch = [
        pltpu.VMEM((CH,), jnp.int32),
        pltpu.VMEM((CH,), jnp.int32),
        pltpu.VMEM((CH, H), jnp.float32),
        pltpu.VMEM((8, H), jnp.float32),
        pltpu.VMEM_SHARED((NPAD, H), jnp.float32),
        pltpu.SemaphoreType.DMA,
    ]
    if with_counts:
        out_type.append(jax.ShapeDtypeStruct((NC, NPAD), jnp.float32))
        scratch += [
            pltpu.VMEM((CH,), jnp.float32),
            pltpu.VMEM((SR,), jnp.float32),
            pltpu.VMEM_SHARED((NPAD,), jnp.float32),
        ]

    def body(h_hbm, srcR_hbm, dstR_hbm, psum_hbm, *rest):
        if with_counts:
            (pcnt_hbm, src_v, dst_v, rows_v, zrow_v, acc_sh, sem,
             ones_v, zcnt_v, cnt_sh) = rest
        else:
            (src_v, dst_v, rows_v, zrow_v, acc_sh, sem) = rest
        cid = lax.axis_index("c")
        sid = lax.axis_index("s")
        wid = cid * NS + sid

        _fill_zero_rows(zrow_v)
        nbase = sid * SR

        def zstep(t, carry):
            pltpu.sync_copy(zrow_v, acc_sh.at[pl.ds(nbase + t * 8, 8)])
            return carry
        lax.fori_loop(0, SR // 8, zstep, 0)
        if with_counts:
            z16 = jnp.zeros((16,), jnp.float32)
            for k in range(SR // 16):
                zcnt_v[pl.ds(k * 16, 16)] = z16
            o16 = jnp.ones((16,), jnp.float32)
            for k in range(CH // 16):
                ones_v[pl.ds(k * 16, 16)] = o16
            pltpu.sync_copy(zcnt_v, cnt_sh.at[pl.ds(nbase, SR)])
        plsc.subcore_barrier()

        ebase = wid * RPW

        def step(j, carry):
            r = ebase + j
            pltpu.sync_copy(srcR_hbm.at[r], src_v)
            pltpu.sync_copy(dstR_hbm.at[r], dst_v)
            pltpu.async_copy(h_hbm.at[src_v], rows_v, sem).wait()
            pltpu.sync_copy(rows_v, acc_sh.at[dst_v], add=True)
            if with_counts:
                pltpu.sync_copy(ones_v, cnt_sh.at[dst_v], add=True)
            return carry
        lax.fori_loop(0, RPW, step, 0)
        plsc.subcore_barrier()

        pltpu.sync_copy(acc_sh.at[pl.ds(nbase, SR)],
                        psum_hbm.at[cid, pl.ds(nbase, SR)])
        if with_counts:
            pltpu.sync_copy(cnt_sh.at[pl.ds(nbase, SR)],
                            pcnt_hbm.at[cid, pl.ds(nbase, SR)])

    return pl.kernel(body, out_type=out_type, mesh=_mesh,
                     scratch_types=scratch)


_agg_counts = _make_agg(True)
_agg = _make_agg(False)


def _proj_body(x_ref, w_ref, b_ref, o_ref):
    o_ref[...] = jax.nn.relu(
        jnp.dot(x_ref[...], w_ref[...], preferred_element_type=jnp.float32)
        + b_ref[...])


_proj = pl.pallas_call(
    _proj_body, out_shape=jax.ShapeDtypeStruct((N, H), jnp.float32))


def _agg_bn(p_ref, c_ref, hprev_ref, w_ref, b_ref, g_ref, be_ref):
    inv = 1.0 / (c_ref[0, :N, :] + c_ref[1, :N, :] + 1.0)
    agg = (p_ref[0, :N, :] + p_ref[1, :N, :] + hprev_ref[...]) * inv
    z = jnp.dot(agg, w_ref[...], preferred_element_type=jnp.float32) + b_ref[...]
    mu = jnp.mean(z, axis=0, keepdims=True)
    var = jnp.mean((z - mu) ** 2, axis=0, keepdims=True)
    return jax.nn.relu((z - mu) * lax.rsqrt(var + EPS) * g_ref[...] + be_ref[...])


def _layer0_body(p_ref, c_ref, hprev_ref, w_ref, b_ref, g_ref, be_ref, o_ref):
    o_ref[...] = _agg_bn(p_ref, c_ref, hprev_ref, w_ref, b_ref, g_ref, be_ref)


_layer0 = pl.pallas_call(
    _layer0_body, out_shape=jax.ShapeDtypeStruct((N, H), jnp.float32))


def _layer1_body(p_ref, c_ref, hprev_ref, skip_ref, w_ref, b_ref, g_ref,
                 be_ref, o_ref):
    o_ref[...] = _agg_bn(p_ref, c_ref, hprev_ref, w_ref, b_ref, g_ref,
                         be_ref) + skip_ref[...]


_layer1 = pl.pallas_call(
    _layer1_body, out_shape=jax.ShapeDtypeStruct((N, H), jnp.float32))


def _final_body(p_ref, c_ref, hprev_ref, h0_ref, h1_ref, w_ref, b_ref, g_ref,
                be_ref, lw_ref, w1_ref, b1_ref, w2_ref, b2_ref, w3_ref,
                b3_ref, o_ref):
    h3 = _agg_bn(p_ref, c_ref, hprev_ref, w_ref, b_ref, g_ref,
                 be_ref) + h0_ref[...]
    lw = lw_ref[...]
    e = jnp.exp(lw - jnp.max(lw, axis=0, keepdims=True))
    wsm = e / jnp.sum(e, axis=0, keepdims=True)
    hc = (h1_ref[...] * wsm[0:1, :] + hprev_ref[...] * wsm[1:2, :]
          + h3 * wsm[2:3, :])
    a1 = jax.nn.relu(
        jnp.dot(hc, w1_ref[...], preferred_element_type=jnp.float32)
        + b1_ref[...])
    a2 = jax.nn.relu(
        jnp.dot(a1, w2_ref[...], preferred_element_type=jnp.float32)
        + b2_ref[...])
    o_ref[...] = (jnp.dot(a2, w3_ref[...], preferred_element_type=jnp.float32)
                  + b3_ref[...])


_final = pl.pallas_call(
    _final_body, out_shape=jax.ShapeDtypeStruct((N, 8), jnp.float32))


def kernel(x, edge_index, Wp, bp, Wconv, bconv, gamma, beta, layer_weights,
           Wc1, bc1, Wc2, bc2, Wc3, bc3):
    pad = EEPAD - E
    ar = jnp.arange(pad, dtype=jnp.int32)
    srcR = jnp.concatenate([edge_index[0], ar % N]).reshape(ROWS, CH)
    dstR = jnp.concatenate([edge_index[1], N + (ar % (NPAD - N))]).reshape(ROWS, CH)

    b2d = lambda v: v.reshape(1, -1)

    h0 = _proj(x, Wp.T, b2d(bp))
    p1, cnt = _agg_counts(h0, srcR, dstR)
    c3 = cnt.reshape(NC, NPAD, 1)
    h1 = _layer0(p1, c3, h0, Wconv[0].T, b2d(bconv[0]), b2d(gamma[0]),
                 b2d(beta[0]))
    (p2,) = _agg(h1, srcR, dstR)
    h2 = _layer1(p2, c3, h1, h0, Wconv[1].T, b2d(bconv[1]), b2d(gamma[1]),
                 b2d(beta[1]))
    (p3,) = _agg(h2, srcR, dstR)
    lwb = jnp.broadcast_to(layer_weights.reshape(3, 1), (3, H))
    Wc3p = jnp.zeros((8, H // 2), jnp.float32).at[:C].set(Wc3)
    bc3p = jnp.zeros((1, 8), jnp.float32).at[0, :C].set(bc3)
    out8 = _final(p3, c3, h2, h0, h1, Wconv[2].T, b2d(bconv[2]),
                  b2d(gamma[2]), b2d(beta[2]), lwb, Wc1.T, b2d(bc1), Wc2.T,
                  b2d(bc2), Wc3p.T, bc3p)
    return out8[:, :C]


# speedup vs baseline: 14.4022x; 1.5922x over previous
import functools

import jax
import jax.numpy as jnp
from jax import lax
from jax.experimental import pallas as pl
from jax.experimental.pallas import tpu as pltpu
from jax.experimental.pallas import tpu_sc as plsc

N = 10000
E = 320000
D = 128
H = 128
C = 2
EPS = 1e-5

NC = 2
NS = 16
NW = NC * NS

NPAD = 10240
SR = NPAD // NS
CH = 128
EEPAD = 327680
ROWS = EEPAD // CH
RPW = ROWS // NW
GK = 16

_mesh = plsc.VectorSubcoreMesh(
    core_axis_name="c", subcore_axis_name="s", num_cores=NC, num_subcores=NS)


def _fill_zero_rows(zrow_v):
    z16 = jnp.zeros((16,), jnp.float32)
    for r in range(8):
        for k in range(8):
            zrow_v[r, pl.ds(k * 16, 16)] = z16


def _make_agg(with_counts):
    out_type = [jax.ShapeDtypeStruct((NC, NPAD, H), jnp.float32)]
    scratch = [
        pltpu.VMEM((GK, CH), jnp.int32),
        pltpu.VMEM((GK, CH), jnp.int32),
        pltpu.VMEM((CH, H), jnp.float32),
        pltpu.VMEM((CH, H), jnp.float32),
        pltpu.VMEM((8, H), jnp.float32),
        pltpu.VMEM_SHARED((NPAD, H), jnp.float32),
        pltpu.SemaphoreType.DMA,
        pltpu.SemaphoreType.DMA,
    ]
    if with_counts:
        out_type.append(jax.ShapeDtypeStruct((NC, NPAD), jnp.float32))
        scratch += [
            pltpu.VMEM((CH,), jnp.float32),
            pltpu.VMEM((SR,), jnp.float32),
            pltpu.VMEM_SHARED((NPAD,), jnp.float32),
        ]

    def body(h_hbm, srcR_hbm, dstR_hbm, psum_hbm, *rest):
        if with_counts:
            (pcnt_hbm, src_vB, dst_vB, rows_a, rows_b, zrow_v, acc_sh,
             sem_a, sem_b, ones_v, zcnt_v, cnt_sh) = rest
        else:
            (src_vB, dst_vB, rows_a, rows_b, zrow_v, acc_sh,
             sem_a, sem_b) = rest
        cid = lax.axis_index("c")
        sid = lax.axis_index("s")
        wid = cid * NS + sid
        ebase = wid * RPW

        _fill_zero_rows(zrow_v)
        nbase = sid * SR

        def zstep(t, carry):
            pltpu.sync_copy(zrow_v, acc_sh.at[pl.ds(nbase + t * 8, 8)])
            return carry
        lax.fori_loop(0, SR // 8, zstep, 0)
        if with_counts:
            z16 = jnp.zeros((16,), jnp.float32)
            for k in range(SR // 16):
                zcnt_v[pl.ds(k * 16, 16)] = z16
            o16 = jnp.ones((16,), jnp.float32)
            for k in range(CH // 16):
                ones_v[pl.ds(k * 16, 16)] = o16
            pltpu.sync_copy(zcnt_v, cnt_sh.at[pl.ds(nbase, SR)])
        plsc.subcore_barrier()

        def scat(k, rows_v):
            pltpu.sync_copy(rows_v, acc_sh.at[dst_vB.at[k]], add=True)
            if with_counts:
                pltpu.sync_copy(ones_v, cnt_sh.at[dst_vB.at[k]], add=True)

        def group(g, carry):
            gbase = ebase + g * GK
            pltpu.sync_copy(srcR_hbm.at[pl.ds(gbase, GK)], src_vB)
            pltpu.sync_copy(dstR_hbm.at[pl.ds(gbase, GK)], dst_vB)
            pltpu.async_copy(h_hbm.at[src_vB.at[0]], rows_a, sem_a)

            def step(t, c2):
                k0 = 2 * t
                pltpu.make_async_copy(h_hbm.at[src_vB.at[k0]], rows_a,
                                      sem_a).wait()
                pltpu.async_copy(h_hbm.at[src_vB.at[k0 + 1]], rows_b, sem_b)
                scat(k0, rows_a)
                pltpu.make_async_copy(h_hbm.at[src_vB.at[k0 + 1]], rows_b,
                                      sem_b).wait()
                kn = lax.select(k0 + 2 < GK, k0 + 2, 0)
                pltpu.async_copy(h_hbm.at[src_vB.at[kn]], rows_a, sem_a)
                scat(k0 + 1, rows_b)
                return c2
            lax.fori_loop(0, GK // 2, step, 0)
            pltpu.make_async_copy(h_hbm.at[src_vB.at[0]], rows_a,
                                  sem_a).wait()
            return carry
        lax.fori_loop(0, RPW // GK, group, 0)
        plsc.subcore_barrier()

        pltpu.sync_copy(acc_sh.at[pl.ds(nbase, SR)],
                        psum_hbm.at[cid, pl.ds(nbase, SR)])
        if with_counts:
            pltpu.sync_copy(cnt_sh.at[pl.ds(nbase, SR)],
                            pcnt_hbm.at[cid, pl.ds(nbase, SR)])

    return pl.kernel(body, out_type=out_type, mesh=_mesh,
                     scratch_types=scratch)


_agg_counts = _make_agg(True)
_agg = _make_agg(False)


def _proj_body(x_ref, w_ref, b_ref, o_ref):
    o_ref[...] = jax.nn.relu(
        jnp.dot(x_ref[...], w_ref[...], preferred_element_type=jnp.float32)
        + b_ref[...])


_proj = pl.pallas_call(
    _proj_body, out_shape=jax.ShapeDtypeStruct((N, H), jnp.float32))


def _agg_bn(p_ref, c_ref, hprev_ref, w_ref, b_ref, g_ref, be_ref):
    inv = 1.0 / (c_ref[0, :N, :] + c_ref[1, :N, :] + 1.0)
    agg = (p_ref[0, :N, :] + p_ref[1, :N, :] + hprev_ref[...]) * inv
    z = jnp.dot(agg, w_ref[...], preferred_element_type=jnp.float32) + b_ref[...]
    mu = jnp.mean(z, axis=0, keepdims=True)
    var = jnp.mean((z - mu) ** 2, axis=0, keepdims=True)
    return jax.nn.relu((z - mu) * lax.rsqrt(var + EPS) * g_ref[...] + be_ref[...])


def _layer0_body(p_ref, c_ref, hprev_ref, w_ref, b_ref, g_ref, be_ref, o_ref):
    o_ref[...] = _agg_bn(p_ref, c_ref, hprev_ref, w_ref, b_ref, g_ref, be_ref)


_layer0 = pl.pallas_call(
    _layer0_body, out_shape=jax.ShapeDtypeStruct((N, H), jnp.float32))


def _layer1_body(p_ref, c_ref, hprev_ref, skip_ref, w_ref, b_ref, g_ref,
                 be_ref, o_ref):
    o_ref[...] = _agg_bn(p_ref, c_ref, hprev_ref, w_ref, b_ref, g_ref,
                         be_ref) + skip_ref[...]


_layer1 = pl.pallas_call(
    _layer1_body, out_shape=jax.ShapeDtypeStruct((N, H), jnp.float32))


def _final_body(p_ref, c_ref, hprev_ref, h0_ref, h1_ref, w_ref, b_ref, g_ref,
                be_ref, lw_ref, w1_ref, b1_ref, w2_ref, b2_ref, w3_ref,
                b3_ref, o_ref):
    h3 = _agg_bn(p_ref, c_ref, hprev_ref, w_ref, b_ref, g_ref,
                 be_ref) + h0_ref[...]
    lw = lw_ref[...]
    e = jnp.exp(lw - jnp.max(lw, axis=0, keepdims=True))
    wsm = e / jnp.sum(e, axis=0, keepdims=True)
    hc = (h1_ref[...] * wsm[0:1, :] + hprev_ref[...] * wsm[1:2, :]
          + h3 * wsm[2:3, :])
    a1 = jax.nn.relu(
        jnp.dot(hc, w1_ref[...], preferred_element_type=jnp.float32)
        + b1_ref[...])
    a2 = jax.nn.relu(
        jnp.dot(a1, w2_ref[...], preferred_element_type=jnp.float32)
        + b2_ref[...])
    o_ref[...] = (jnp.dot(a2, w3_ref[...], preferred_element_type=jnp.float32)
                  + b3_ref[...])


_final = pl.pallas_call(
    _final_body, out_shape=jax.ShapeDtypeStruct((N, 8), jnp.float32))


def kernel(x, edge_index, Wp, bp, Wconv, bconv, gamma, beta, layer_weights,
           Wc1, bc1, Wc2, bc2, Wc3, bc3):
    pad = EEPAD - E
    ar = jnp.arange(pad, dtype=jnp.int32)
    srcR = jnp.concatenate([edge_index[0], ar % N]).reshape(ROWS, CH)
    dstR = jnp.concatenate([edge_index[1], N + (ar % (NPAD - N))]).reshape(ROWS, CH)

    b2d = lambda v: v.reshape(1, -1)

    h0 = _proj(x, Wp.T, b2d(bp))
    p1, cnt = _agg_counts(h0, srcR, dstR)
    c3 = cnt.reshape(NC, NPAD, 1)
    h1 = _layer0(p1, c3, h0, Wconv[0].T, b2d(bconv[0]), b2d(gamma[0]),
                 b2d(beta[0]))
    (p2,) = _agg(h1, srcR, dstR)
    h2 = _layer1(p2, c3, h1, h0, Wconv[1].T, b2d(bconv[1]), b2d(gamma[1]),
                 b2d(beta[1]))
    (p3,) = _agg(h2, srcR, dstR)
    lwb = jnp.broadcast_to(layer_weights.reshape(3, 1), (3, H))
    Wc3p = jnp.zeros((8, H // 2), jnp.float32).at[:C].set(Wc3)
    bc3p = jnp.zeros((1, 8), jnp.float32).at[0, :C].set(bc3)
    out8 = _final(p3, c3, h2, h0, h1, Wconv[2].T, b2d(bconv[2]),
                  b2d(gamma[2]), b2d(beta[2]), lwb, Wc1.T, b2d(bc1), Wc2.T,
                  b2d(bc2), Wc3p.T, bc3p)
    return out8[:, :C]


# speedup vs baseline: 18.1328x; 1.2590x over previous
import functools

import jax
import jax.numpy as jnp
from jax import lax
from jax.experimental import pallas as pl
from jax.experimental.pallas import tpu as pltpu
from jax.experimental.pallas import tpu_sc as plsc

N = 10000
E = 320000
D = 128
H = 128
C = 2
EPS = 1e-5

NC = 2
NS = 16
NW = NC * NS

NPAD = 10240
SR = NPAD // NS
CH = 64
RPW = 160
GW = 80
NT = (GW - 3) // 3
EEPAD = NW * RPW * CH
ROWS = EEPAD // CH

_mesh = plsc.VectorSubcoreMesh(
    core_axis_name="c", subcore_axis_name="s", num_cores=NC, num_subcores=NS)


def _fill_zero_rows(zrow_v):
    z16 = jnp.zeros((16,), jnp.float32)
    for r in range(8):
        for k in range(8):
            zrow_v[r, pl.ds(k * 16, 16)] = z16


def _make_agg(with_counts):
    out_type = [jax.ShapeDtypeStruct((NC, NPAD, H), jnp.float32)]
    scratch = [
        pltpu.VMEM((GW, CH), jnp.int32),
        pltpu.VMEM((GW, CH), jnp.int32),
        pltpu.VMEM((CH, H), jnp.float32),
        pltpu.VMEM((CH, H), jnp.float32),
        pltpu.VMEM((CH, H), jnp.float32),
        pltpu.VMEM((8, H), jnp.float32),
        pltpu.VMEM_SHARED((NPAD, H), jnp.float32),
        pltpu.SemaphoreType.DMA,
        pltpu.SemaphoreType.DMA,
        pltpu.SemaphoreType.DMA,
        pltpu.SemaphoreType.DMA,
        pltpu.SemaphoreType.DMA,
        pltpu.SemaphoreType.DMA,
    ]
    if with_counts:
        out_type.append(jax.ShapeDtypeStruct((NC, NPAD), jnp.float32))
        scratch += [
            pltpu.VMEM((CH,), jnp.float32),
            pltpu.VMEM((SR,), jnp.float32),
            pltpu.VMEM_SHARED((NPAD,), jnp.float32),
        ]

    def body(h_hbm, srcR_hbm, dstR_hbm, psum_hbm, *rest):
        if with_counts:
            (pcnt_hbm, src_vB, dst_vB, r0, r1, r2, zrow_v, acc_sh,
             sg0, sg1, sg2, ss0, ss1, ss2, ones_v, zcnt_v, cnt_sh) = rest
        else:
            (src_vB, dst_vB, r0, r1, r2, zrow_v, acc_sh,
             sg0, sg1, sg2, ss0, ss1, ss2) = rest
        rows = (r0, r1, r2)
        sg = (sg0, sg1, sg2)
        ss = (ss0, ss1, ss2)
        cid = lax.axis_index("c")
        sid = lax.axis_index("s")
        wid = cid * NS + sid
        ebase = wid * RPW

        _fill_zero_rows(zrow_v)
        nbase = sid * SR

        def zstep(t, carry):
            pltpu.sync_copy(zrow_v, acc_sh.at[pl.ds(nbase + t * 8, 8)])
            return carry
        lax.fori_loop(0, SR // 8, zstep, 0)
        if with_counts:
            z16 = jnp.zeros((16,), jnp.float32)
            for k in range(SR // 16):
                zcnt_v[pl.ds(k * 16, 16)] = z16
            o16 = jnp.ones((16,), jnp.float32)
            for k in range(CH // 16):
                ones_v[pl.ds(k * 16, 16)] = o16
            pltpu.sync_copy(zcnt_v, cnt_sh.at[pl.ds(nbase, SR)])
        plsc.subcore_barrier()

        def gat(j, b):
            pltpu.async_copy(h_hbm.at[src_vB.at[j]], rows[b], sg[b])

        def wait_g(j, b):
            pltpu.make_async_copy(h_hbm.at[src_vB.at[j]], rows[b],
                                  sg[b]).wait()

        def scat(j, b):
            pltpu.async_copy(rows[b], acc_sh.at[dst_vB.at[j]], ss[b],
                             add=True)
            if with_counts:
                pltpu.sync_copy(ones_v, cnt_sh.at[dst_vB.at[j]], add=True)

        def wait_s(b):
            pltpu.make_async_copy(rows[b], acc_sh.at[dst_vB.at[0]],
                                  ss[b]).wait()

        def iter_body(j, b, first, prefetch):
            bp = (b + 2) % 3
            if prefetch:
                if not first:
                    wait_s(bp)
                gat(j + 2, bp)
            wait_g(j, b)
            scat(j, b)

        for g in range(RPW // GW):
            gbase = ebase + g * GW
            pltpu.sync_copy(srcR_hbm.at[pl.ds(gbase, GW)], src_vB)
            pltpu.sync_copy(dstR_hbm.at[pl.ds(gbase, GW)], dst_vB)
            gat(0, 0)
            gat(1, 1)
            iter_body(0, 0, True, True)

            def triple(t, carry):
                j = 3 * t + 1
                iter_body(j, 1, False, True)
                iter_body(j + 1, 2, False, True)
                iter_body(j + 2, 0, False, True)
                return carry
            lax.fori_loop(0, NT, triple, 0)
            for j in range(3 * NT + 1, GW):
                iter_body(j, j % 3, False, j + 2 < GW)
            wait_s(0)
            wait_s(1)
            wait_s(2)
        plsc.subcore_barrier()

        pltpu.sync_copy(acc_sh.at[pl.ds(nbase, SR)],
                        psum_hbm.at[cid, pl.ds(nbase, SR)])
        if with_counts:
            pltpu.sync_copy(cnt_sh.at[pl.ds(nbase, SR)],
                            pcnt_hbm.at[cid, pl.ds(nbase, SR)])

    return pl.kernel(body, out_type=out_type, mesh=_mesh,
                     scratch_types=scratch)


_agg_counts = _make_agg(True)
_agg = _make_agg(False)


def _proj_body(x_ref, w_ref, b_ref, o_ref):
    o_ref[...] = jax.nn.relu(
        jnp.dot(x_ref[...], w_ref[...], preferred_element_type=jnp.float32)
        + b_ref[...])


_proj = pl.pallas_call(
    _proj_body, out_shape=jax.ShapeDtypeStruct((N, H), jnp.float32))


def _agg_bn(p_ref, c_ref, hprev_ref, w_ref, b_ref, g_ref, be_ref):
    inv = 1.0 / (c_ref[0, :N, :] + c_ref[1, :N, :] + 1.0)
    agg = (p_ref[0, :N, :] + p_ref[1, :N, :] + hprev_ref[...]) * inv
    z = jnp.dot(agg, w_ref[...], preferred_element_type=jnp.float32) + b_ref[...]
    mu = jnp.mean(z, axis=0, keepdims=True)
    var = jnp.mean((z - mu) ** 2, axis=0, keepdims=True)
    return jax.nn.relu((z - mu) * lax.rsqrt(var + EPS) * g_ref[...] + be_ref[...])


def _layer0_body(p_ref, c_ref, hprev_ref, w_ref, b_ref, g_ref, be_ref, o_ref):
    o_ref[...] = _agg_bn(p_ref, c_ref, hprev_ref, w_ref, b_ref, g_ref, be_ref)


_layer0 = pl.pallas_call(
    _layer0_body, out_shape=jax.ShapeDtypeStruct((N, H), jnp.float32))


def _layer1_body(p_ref, c_ref, hprev_ref, skip_ref, w_ref, b_ref, g_ref,
                 be_ref, o_ref):
    o_ref[...] = _agg_bn(p_ref, c_ref, hprev_ref, w_ref, b_ref, g_ref,
                         be_ref) + skip_ref[...]


_layer1 = pl.pallas_call(
    _layer1_body, out_shape=jax.ShapeDtypeStruct((N, H), jnp.float32))


def _final_body(p_ref, c_ref, hprev_ref, h0_ref, h1_ref, w_ref, b_ref, g_ref,
                be_ref, lw_ref, w1_ref, b1_ref, w2_ref, b2_ref, w3_ref,
                b3_ref, o_ref):
    h3 = _agg_bn(p_ref, c_ref, hprev_ref, w_ref, b_ref, g_ref,
                 be_ref) + h0_ref[...]
    lw = lw_ref[...]
    e = jnp.exp(lw - jnp.max(lw, axis=0, keepdims=True))
    wsm = e / jnp.sum(e, axis=0, keepdims=True)
    hc = (h1_ref[...] * wsm[0:1, :] + hprev_ref[...] * wsm[1:2, :]
          + h3 * wsm[2:3, :])
    a1 = jax.nn.relu(
        jnp.dot(hc, w1_ref[...], preferred_element_type=jnp.float32)
        + b1_ref[...])
    a2 = jax.nn.relu(
        jnp.dot(a1, w2_ref[...], preferred_element_type=jnp.float32)
        + b2_ref[...])
    o_ref[...] = (jnp.dot(a2, w3_ref[...], preferred_element_type=jnp.float32)
                  + b3_ref[...])


_final = pl.pallas_call(
    _final_body, out_shape=jax.ShapeDtypeStruct((N, 8), jnp.float32))


def kernel(x, edge_index, Wp, bp, Wconv, bconv, gamma, beta, layer_weights,
           Wc1, bc1, Wc2, bc2, Wc3, bc3):
    pad = EEPAD - E
    ar = jnp.arange(pad, dtype=jnp.int32)
    srcR = jnp.concatenate([edge_index[0], ar % N]).reshape(ROWS, CH)
    dstR = jnp.concatenate([edge_index[1], N + (ar % (NPAD - N))]).reshape(ROWS, CH)

    b2d = lambda v: v.reshape(1, -1)

    h0 = _proj(x, Wp.T, b2d(bp))
    p1, cnt = _agg_counts(h0, srcR, dstR)
    c3 = cnt.reshape(NC, NPAD, 1)
    h1 = _layer0(p1, c3, h0, Wconv[0].T, b2d(bconv[0]), b2d(gamma[0]),
                 b2d(beta[0]))
    (p2,) = _agg(h1, srcR, dstR)
    h2 = _layer1(p2, c3, h1, h0, Wconv[1].T, b2d(bconv[1]), b2d(gamma[1]),
                 b2d(beta[1]))
    (p3,) = _agg(h2, srcR, dstR)
    lwb = jnp.broadcast_to(layer_weights.reshape(3, 1), (3, H))
    Wc3p = jnp.zeros((8, H // 2), jnp.float32).at[:C].set(Wc3)
    bc3p = jnp.zeros((1, 8), jnp.float32).at[0, :C].set(bc3)
    out8 = _final(p3, c3, h2, h0, h1, Wconv[2].T, b2d(bconv[2]),
                  b2d(gamma[2]), b2d(beta[2]), lwb, Wc1.T, b2d(bc1), Wc2.T,
                  b2d(bc2), Wc3p.T, bc3p)
    return out8[:, :C]


# speedup vs baseline: 18.1824x; 1.0027x over previous
import jax
import jax.numpy as jnp
from jax import lax
from jax.experimental import pallas as pl
from jax.experimental.pallas import tpu as pltpu
from jax.experimental.pallas import tpu_sc as plsc

N = 10000
E = 320000
D = 128
H = 128
C = 2
EPS = 1e-5

NC = 2
NS = 16
NW = NC * NS

NPAD = 10240
SR = NPAD // NS
CH = 64
RPW = 160
GW = 80
NT = (GW - 3) // 3
EEPAD = NW * RPW * CH
ROWS = EEPAD // CH

_mesh = plsc.VectorSubcoreMesh(
    core_axis_name="c", subcore_axis_name="s", num_cores=NC, num_subcores=NS)


def _make_agg(with_counts):
    out_type = [jax.ShapeDtypeStruct((NC, NPAD, H), jnp.float32)]
    scratch = [
        pltpu.VMEM((GW, CH), jnp.int32),
        pltpu.VMEM((GW, CH), jnp.int32),
        pltpu.VMEM((CH, H), jnp.float32),
        pltpu.VMEM((CH, H), jnp.float32),
        pltpu.VMEM((CH, H), jnp.float32),
        pltpu.VMEM_SHARED((NPAD, H), jnp.float32),
        pltpu.SemaphoreType.DMA,
        pltpu.SemaphoreType.DMA,
        pltpu.SemaphoreType.DMA,
        pltpu.SemaphoreType.DMA,
        pltpu.SemaphoreType.DMA,
        pltpu.SemaphoreType.DMA,
    ]
    if with_counts:
        out_type.append(jax.ShapeDtypeStruct((NC, NPAD), jnp.float32))
        scratch += [
            pltpu.VMEM((CH,), jnp.float32),
            pltpu.VMEM((SR,), jnp.float32),
            pltpu.VMEM_SHARED((NPAD,), jnp.float32),
        ]

    def body(h_hbm, srcR_hbm, dstR_hbm, zrows_hbm, psum_hbm, *rest):
        if with_counts:
            (pcnt_hbm, src_vB, dst_vB, r0, r1, r2, acc_sh,
             sg0, sg1, sg2, ss0, ss1, ss2, ones_v, zcnt_v, cnt_sh) = rest
        else:
            (src_vB, dst_vB, r0, r1, r2, acc_sh,
             sg0, sg1, sg2, ss0, ss1, ss2) = rest
        rows = (r0, r1, r2)
        sg = (sg0, sg1, sg2)
        ss = (ss0, ss1, ss2)
        cid = lax.axis_index("c")
        sid = lax.axis_index("s")
        wid = cid * NS + sid
        ebase = wid * RPW
        nbase = sid * SR

        @pl.when(cid == 0)
        def _():
            pltpu.sync_copy(h_hbm.at[pl.ds(nbase, SR)],
                            acc_sh.at[pl.ds(nbase, SR)])

        @pl.when(cid != 0)
        def _():
            pltpu.sync_copy(zrows_hbm, acc_sh.at[pl.ds(nbase, SR)])

        if with_counts:
            z16 = jnp.zeros((16,), jnp.float32)
            for k in range(SR // 16):
                zcnt_v[pl.ds(k * 16, 16)] = z16
            o16 = jnp.ones((16,), jnp.float32)
            for k in range(CH // 16):
                ones_v[pl.ds(k * 16, 16)] = o16
            pltpu.sync_copy(zcnt_v, cnt_sh.at[pl.ds(nbase, SR)])
        plsc.subcore_barrier()

        def gat(j, b):
            pltpu.async_copy(h_hbm.at[src_vB.at[j]], rows[b], sg[b])

        def wait_g(j, b):
            pltpu.make_async_copy(h_hbm.at[src_vB.at[j]], rows[b],
                                  sg[b]).wait()

        def scat(j, b):
            pltpu.async_copy(rows[b], acc_sh.at[dst_vB.at[j]], ss[b],
                             add=True)
            if with_counts:
                pltpu.sync_copy(ones_v, cnt_sh.at[dst_vB.at[j]], add=True)

        def wait_s(b):
            pltpu.make_async_copy(rows[b], acc_sh.at[dst_vB.at[0]],
                                  ss[b]).wait()

        def iter_body(j, b, first, prefetch):
            bp = (b + 2) % 3
            if prefetch:
                if not first:
                    wait_s(bp)
                gat(j + 2, bp)
            wait_g(j, b)
            scat(j, b)

        for g in range(RPW // GW):
            gbase = ebase + g * GW
            pltpu.sync_copy(srcR_hbm.at[pl.ds(gbase, GW)], src_vB)
            pltpu.sync_copy(dstR_hbm.at[pl.ds(gbase, GW)], dst_vB)
            gat(0, 0)
            gat(1, 1)
            iter_body(0, 0, True, True)

            def triple(t, carry):
                j = 3 * t + 1
                iter_body(j, 1, False, True)
                iter_body(j + 1, 2, False, True)
                iter_body(j + 2, 0, False, True)
                return carry
            lax.fori_loop(0, NT, triple, 0)
            for j in range(3 * NT + 1, GW):
                iter_body(j, j % 3, False, j + 2 < GW)
            wait_s(0)
            wait_s(1)
            wait_s(2)
        plsc.subcore_barrier()

        pltpu.sync_copy(acc_sh.at[pl.ds(nbase, SR)],
                        psum_hbm.at[cid, pl.ds(nbase, SR)])
        if with_counts:
            pltpu.sync_copy(cnt_sh.at[pl.ds(nbase, SR)],
                            pcnt_hbm.at[cid, pl.ds(nbase, SR)])

    return pl.kernel(body, out_type=out_type, mesh=_mesh,
                     scratch_types=scratch)


_agg_counts = _make_agg(True)
_agg = _make_agg(False)


def _proj_body(x_ref, w_ref, b_ref, o_ref):
    o_ref[0:N, :] = jax.nn.relu(
        jnp.dot(x_ref[...], w_ref[...], preferred_element_type=jnp.float32)
        + b_ref[...])
    o_ref[N:NPAD, :] = jnp.zeros((NPAD - N, H), jnp.float32)


_proj = pl.pallas_call(
    _proj_body, out_shape=jax.ShapeDtypeStruct((NPAD, H), jnp.float32))


def _agg_bn(p_ref, c_ref, w_ref, b_ref, g_ref, be_ref):
    inv = 1.0 / (c_ref[0] + c_ref[1] + 1.0)
    agg = (p_ref[0] + p_ref[1]) * inv
    z = jnp.dot(agg, w_ref[...], preferred_element_type=jnp.float32) + b_ref[...]
    zs = z[0:N, :]
    mu = jnp.mean(zs, axis=0, keepdims=True)
    var = jnp.mean((zs - mu) ** 2, axis=0, keepdims=True)
    return jax.nn.relu((z - mu) * lax.rsqrt(var + EPS) * g_ref[...] + be_ref[...])


def _layer0_body(p_ref, c_ref, w_ref, b_ref, g_ref, be_ref, o_ref):
    o_ref[...] = _agg_bn(p_ref, c_ref, w_ref, b_ref, g_ref, be_ref)


_layer0 = pl.pallas_call(
    _layer0_body, out_shape=jax.ShapeDtypeStruct((NPAD, H), jnp.float32))


def _layer1_body(p_ref, c_ref, skip_ref, w_ref, b_ref, g_ref, be_ref, o_ref):
    o_ref[...] = _agg_bn(p_ref, c_ref, w_ref, b_ref, g_ref,
                         be_ref) + skip_ref[...]


_layer1 = pl.pallas_call(
    _layer1_body, out_shape=jax.ShapeDtypeStruct((NPAD, H), jnp.float32))


def _final_body(p_ref, c_ref, h0_ref, h1_ref, h2_ref, w_ref, b_ref, g_ref,
                be_ref, lw_ref, w1_ref, b1_ref, w2_ref, b2_ref, w3_ref,
                b3_ref, o_ref):
    h3 = _agg_bn(p_ref, c_ref, w_ref, b_ref, g_ref, be_ref) + h0_ref[...]
    lw = lw_ref[...]
    e = jnp.exp(lw - jnp.max(lw, axis=0, keepdims=True))
    wsm = e / jnp.sum(e, axis=0, keepdims=True)
    hc = (h1_ref[...] * wsm[0:1, :] + h2_ref[...] * wsm[1:2, :]
          + h3 * wsm[2:3, :])
    a1 = jax.nn.relu(
        jnp.dot(hc, w1_ref[...], preferred_element_type=jnp.float32)
        + b1_ref[...])
    a2 = jax.nn.relu(
        jnp.dot(a1, w2_ref[...], preferred_element_type=jnp.float32)
        + b2_ref[...])
    o_ref[...] = (jnp.dot(a2, w3_ref[...], preferred_element_type=jnp.float32)
                  + b3_ref[...])


_final = pl.pallas_call(
    _final_body, out_shape=jax.ShapeDtypeStruct((NPAD, 8), jnp.float32))


def kernel(x, edge_index, Wp, bp, Wconv, bconv, gamma, beta, layer_weights,
           Wc1, bc1, Wc2, bc2, Wc3, bc3):
    pad = EEPAD - E
    ar = jnp.arange(pad, dtype=jnp.int32)
    srcR = jnp.concatenate([edge_index[0], ar % N]).reshape(ROWS, CH)
    dstR = jnp.concatenate([edge_index[1], N + (ar % (NPAD - N))]).reshape(ROWS, CH)
    zrows = jnp.zeros((SR, H), jnp.float32)

    b2d = lambda v: v.reshape(1, -1)

    h0 = _proj(x, Wp.T, b2d(bp))
    p1, cnt = _agg_counts(h0, srcR, dstR, zrows)
    c3 = cnt.reshape(NC, NPAD, 1)
    h1 = _layer0(p1, c3, Wconv[0].T, b2d(bconv[0]), b2d(gamma[0]),
                 b2d(beta[0]))
    (p2,) = _agg(h1, srcR, dstR, zrows)
    h2 = _layer1(p2, c3, h0, Wconv[1].T, b2d(bconv[1]), b2d(gamma[1]),
                 b2d(beta[1]))
    (p3,) = _agg(h2, srcR, dstR, zrows)
    lwb = jnp.broadcast_to(layer_weights.reshape(3, 1), (3, H))
    Wc3p = jnp.zeros((8, H // 2), jnp.float32).at[:C].set(Wc3)
    bc3p = jnp.zeros((1, 8), jnp.float32).at[0, :C].set(bc3)
    out8 = _final(p3, c3, h0, h1, h2, Wconv[2].T, b2d(bconv[2]),
                  b2d(gamma[2]), b2d(beta[2]), lwb, Wc1.T, b2d(bc1), Wc2.T,
                  b2d(bc2), Wc3p.T, bc3p)
    return out8[:N, :C]
